# baseline (speedup 1.0000x reference)
import functools

import jax
import jax.numpy as jnp
from jax import lax
from jax.experimental import pallas as pl
from jax.experimental.pallas import tpu as pltpu
from jax.experimental.pallas import tpu_sc as plsc

TOKENS = 16384
D_MODEL = 4096
N_EXPERTS = 64
BT = 1024

_info = plsc.get_sparse_core_info()
_NW = _info.num_cores * _info.num_subcores
_L = _info.num_lanes


def _matmul_body(x_ref, w_ref, s_ref):
    s_ref[...] = lax.dot_general(
        x_ref[...], w_ref[...], (((1,), (1,)), ((), ())),
        preferred_element_type=jnp.float32,
    )


def _scores_tc(x, W):
    n = x.shape[0]
    grid = (n // BT,)
    return pl.pallas_call(
        _matmul_body,
        grid=grid,
        in_specs=[
            pl.BlockSpec((BT, D_MODEL), lambda i: (i, 0)),
            pl.BlockSpec((N_EXPERTS, D_MODEL), lambda i: (0, 0)),
        ],
        out_specs=pl.BlockSpec((BT, N_EXPERTS), lambda i: (i, 0)),
        out_shape=jax.ShapeDtypeStruct((n, N_EXPERTS), jnp.float32),
    )(x, W)


def _router_sc_make(n_tokens):
    per_w = n_tokens // _NW
    n_groups = per_w // _L
    mesh = plsc.VectorSubcoreMesh(core_axis_name="c", subcore_axis_name="s")

    @functools.partial(
        pl.kernel,
        mesh=mesh,
        compiler_params=pltpu.CompilerParams(needs_layout_passes=False),
        out_type=[
            jax.ShapeDtypeStruct((n_tokens * 2,), jnp.int32),
            jax.ShapeDtypeStruct((n_tokens * 2,), jnp.float32),
        ],
        scratch_types=[
            pltpu.VMEM((per_w * N_EXPERTS,), jnp.float32),
            pltpu.VMEM((per_w * 2,), jnp.int32),
            pltpu.VMEM((per_w * 2,), jnp.float32),
        ],
    )
    def router(scores_hbm, topi_hbm, topv_hbm, sv, ibuf, vbuf):
        wid = lax.axis_index("s") * _info.num_cores + lax.axis_index("c")
        base = wid * per_w
        pltpu.sync_copy(scores_hbm.at[pl.ds(base * N_EXPERTS, per_w * N_EXPERTS)], sv)

        def group(g, carry):
            rows = lax.iota(jnp.int32, _L) + g * _L
            rows64 = rows * N_EXPERTS
            rows2 = rows * 2
            m1 = jnp.full((_L,), -jnp.inf, jnp.float32)
            m2 = jnp.full((_L,), -jnp.inf, jnp.float32)
            i1 = jnp.zeros((_L,), jnp.int32)
            i2 = jnp.zeros((_L,), jnp.int32)
            for e in range(N_EXPERTS):
                cols = jnp.full((_L,), e, jnp.int32)
                v = plsc.load_gather(sv, [rows64 + e])
                gt1 = v > m1
                gt2 = v > m2
                i2 = jnp.where(gt1, i1, jnp.where(gt2, cols, i2))
                m2 = jnp.where(gt1, m1, jnp.where(gt2, v, m2))
                i1 = jnp.where(gt1, cols, i1)
                m1 = jnp.where(gt1, v, m1)
            e2 = jnp.exp(m2 - m1)
            v1 = 1.0 / (1.0 + e2)
            v2 = 1.0 - v1
            plsc.store_scatter(ibuf, [rows2], i1)
            plsc.store_scatter(ibuf, [rows2 + 1], i2)
            plsc.store_scatter(vbuf, [rows2], v1)
            plsc.store_scatter(vbuf, [rows2 + 1], v2)
            return carry

        lax.fori_loop(0, n_groups, group, 0)
        pltpu.sync_copy(ibuf, topi_hbm.at[pl.ds(base * 2, per_w * 2)])
        pltpu.sync_copy(vbuf, topv_hbm.at[pl.ds(base * 2, per_w * 2)])

    return router


N_CHUNKS = 4
_CHUNK = TOKENS // N_CHUNKS
_router_sc = _router_sc_make(_CHUNK)


def kernel(x, W):
    topi_parts = []
    topv_parts = []
    for c in range(N_CHUNKS):
        xc = lax.slice_in_dim(x, c * _CHUNK, (c + 1) * _CHUNK, axis=0)
        scores = _scores_tc(xc, W)
        topi_flat, topv_flat = _router_sc(scores.reshape(-1))
        topi_parts.append(topi_flat.reshape(_CHUNK, 2))
        topv_parts.append(topv_flat.reshape(_CHUNK, 2))
    return (jnp.concatenate(topi_parts, 0), jnp.concatenate(topv_parts, 0))


# speedup vs baseline: 3.2031x; 3.2031x over previous
import jax
import jax.numpy as jnp
from jax import lax
from jax.experimental import pallas as pl

TOKENS = 16384
D_MODEL = 4096
N_EXPERTS = 64
BT = 1024


def _router_body(x_ref, w_ref, topi_ref, topv_ref):
    x = x_ref[...]
    w = w_ref[...]
    scores = lax.dot_general(
        x, w, (((1,), (1,)), ((), ())), preferred_element_type=jnp.float32
    )

    col = lax.broadcasted_iota(jnp.int32, scores.shape, 1)
    m1 = jnp.max(scores, axis=1, keepdims=True)
    i1 = jnp.min(jnp.where(scores == m1, col, N_EXPERTS), axis=1, keepdims=True)
    masked = jnp.where(col == i1, -jnp.inf, scores)
    m2 = jnp.max(masked, axis=1, keepdims=True)
    i2 = jnp.min(jnp.where(masked == m2, col, N_EXPERTS), axis=1, keepdims=True)

    z = jnp.sum(jnp.exp(scores - m1), axis=1, keepdims=True)
    e2 = jnp.exp(m2 - m1)
    denom = 1.0 + e2 + 1e-9 * z
    v1 = 1.0 / denom
    v2 = e2 / denom

    topi_ref[...] = jnp.concatenate([i1, i2], axis=1)
    topv_ref[...] = jnp.concatenate([v1, v2], axis=1)


def kernel(x, W):
    grid = (TOKENS // BT,)
    topi, topv = pl.pallas_call(
        _router_body,
        grid=grid,
        in_specs=[
            pl.BlockSpec((BT, D_MODEL), lambda i: (i, 0)),
            pl.BlockSpec((N_EXPERTS, D_MODEL), lambda i: (0, 0)),
        ],
        out_specs=[
            pl.BlockSpec((BT, 2), lambda i: (i, 0)),
            pl.BlockSpec((BT, 2), lambda i: (i, 0)),
        ],
        out_shape=[
            jax.ShapeDtypeStruct((TOKENS, 2), jnp.int32),
            jax.ShapeDtypeStruct((TOKENS, 2), jnp.float32),
        ],
    )(x, W)
    return (topi, topv)
